# baseline (speedup 1.0000x reference)
import functools

import jax
import jax.numpy as jnp
from jax import lax
from jax.experimental import pallas as pl
from jax.experimental.pallas import tpu as pltpu
from jax.experimental.pallas import tpu_sc as plsc

N = 10000
E = 320000
H = 128
R = 16
L = 10

NC = 2
NS = 16
HALFN = N // NC
CHUNK = 128
NCHUNK = -(-E // (NS * CHUNK))
E_PAD = NS * NCHUNK * CHUNK
AGG_ROWS = 5120
ROWS_PER_SUB = AGG_ROWS // NS

BN = 1000
GRID = N // BN


def _sc_body(hall, idx2, out, idxring, rows, aggs, gsems, ssems, isems):
    cid = lax.axis_index("c")
    sid = lax.axis_index("s")

    def zero_row(r, _):
        for j in range(H // 16):
            rows[0, r, pl.ds(j * 16, 16)] = jnp.zeros((16,), jnp.float32)
        return 0

    lax.fori_loop(0, CHUNK, zero_row, 0)
    base = sid * ROWS_PER_SUB
    off = 0
    while off < ROWS_PER_SUB:
        step = min(CHUNK, ROWS_PER_SUB - off)
        pltpu.sync_copy(rows.at[0, pl.ds(0, step)],
                        aggs.at[pl.ds(base + off, step)])
        off += step

    plsc.subcore_barrier()

    for k in range(4):
        pltpu.async_copy(idx2.at[cid, sid, k], idxring.at[k], isems.at[k])
    for k in range(2):
        pltpu.make_async_copy(idx2.at[cid, sid, k], idxring.at[k],
                              isems.at[k]).wait()
        pltpu.async_copy(hall.at[idxring.at[k, 0]], rows.at[k], gsems.at[k])

    def chunk_step(c, _):
        si = lax.rem(c, 8)
        sr = lax.rem(c, 4)
        sg = lax.rem(c, 2)
        pltpu.make_async_copy(hall.at[idxring.at[si, 0]], rows.at[sr],
                              gsems.at[sg]).wait()

        @pl.when(c >= 2)
        def _drain_scatter():
            pltpu.make_async_copy(rows.at[lax.rem(c + 2, 4)],
                                  aggs.at[idxring.at[lax.rem(c + 6, 8), 1]],
                                  ssems.at[sg]).wait()

        pltpu.async_copy(rows.at[sr], aggs.at[idxring.at[si, 1]],
                         ssems.at[sg], add=True)

        @pl.when(c + 2 < NCHUNK)
        def _next_gather():
            pltpu.make_async_copy(idx2.at[cid, sid, c + 2],
                                  idxring.at[lax.rem(c + 2, 8)],
                                  isems.at[lax.rem(c + 2, 4)]).wait()
            pltpu.async_copy(hall.at[idxring.at[lax.rem(c + 2, 8), 0]],
                             rows.at[lax.rem(c + 2, 4)], gsems.at[sg])

        @pl.when(c + 4 < NCHUNK)
        def _next_idx():
            pltpu.async_copy(idx2.at[cid, sid, c + 4],
                             idxring.at[lax.rem(c + 4, 8)],
                             isems.at[lax.rem(c + 4, 4)])

        return 0

    lax.fori_loop(0, NCHUNK, chunk_step, 0)

    for c in (NCHUNK - 2, NCHUNK - 1):
        pltpu.make_async_copy(rows.at[c % 4], aggs.at[idxring.at[c % 8, 1]],
                              ssems.at[c % 2]).wait()

    plsc.subcore_barrier()

    pltpu.sync_copy(aggs.at[pl.ds(base, ROWS_PER_SUB)],
                    out.at[cid, pl.ds(base, ROWS_PER_SUB)])


@functools.cache
def _sc_scatter():
    return functools.partial(
        pl.kernel,
        mesh=plsc.VectorSubcoreMesh(core_axis_name="c", subcore_axis_name="s"),
        out_type=jax.ShapeDtypeStruct((NC, AGG_ROWS, H), jnp.float32),
        scratch_types=[
            pltpu.VMEM((8, 2, CHUNK), jnp.int32),
            pltpu.VMEM((4, CHUNK, H), jnp.float32),
            pltpu.VMEM_SHARED((AGG_ROWS, H), jnp.float32),
            pltpu.SemaphoreType.DMA((2,)),
            pltpu.SemaphoreType.DMA((2,)),
            pltpu.SemaphoreType.DMA((4,)),
        ],
    )(_sc_body)


def _relu(v):
    return jnp.maximum(v, 0.0)


def _mm(a, b):
    return jnp.dot(a, b, preferred_element_type=jnp.float32)


def _init_body(x, win, b_in, wrel, h_ref, hall_ref):
    h = _relu(_mm(x[...], win[...]) + b_in[0][None, :])
    h_ref[...] = h
    for r in range(R):
        hall_ref[r] = _mm(h, wrel[r])


def _mlp(hb, agg, wself, b_rel, wu1, b_u1, wu2, b_u2):
    mid = agg[0] + _mm(hb, wself[...]) + b_rel[0][None, :]
    z = _relu(_mm(hb, wu1[0:H, :]) + _mm(mid, wu1[H:2 * H, :])
              + b_u1[0][None, :])
    return _relu(_mm(z, wu2[...]) + b_u2[0][None, :])


def _layer_body(h, agg, wself, b_rel, wu1, b_u1, wu2, b_u2, wrel,
                h_ref, hall_ref):
    out = _mlp(h[...], agg, wself, b_rel, wu1, b_u1, wu2, b_u2)
    h_ref[...] = out
    for r in range(R):
        hall_ref[r] = _mm(out, wrel[r])


def _final_body(h, agg, wself, b_rel, wu1, b_u1, wu2, b_u2, h_ref):
    h_ref[...] = _mlp(h[...], agg, wself, b_rel, wu1, b_u1, wu2, b_u2)


def _full(shape):
    return pl.BlockSpec(shape, lambda i: tuple(0 for _ in shape))


_ROWBLK = pl.BlockSpec((BN, H), lambda i: (i, 0))
_HALLBLK = pl.BlockSpec((R, BN, H), lambda i: (0, i, 0))
_AGGBLK = pl.BlockSpec((1, BN, H), lambda i: (i // 5, i % 5, 0))
_HALL_SHAPE = jax.ShapeDtypeStruct((R, N, H), jnp.float32)

_MLP_SPECS = [_full((H, H)), _full((1, H)), _full((2 * H, 2 * H)),
              _full((1, 2 * H)), _full((2 * H, H)), _full((1, H))]

_tc_init = pl.pallas_call(
    _init_body,
    grid=(GRID,),
    in_specs=[_ROWBLK, _full((H, H)), _full((1, H)), _full((R, H, H))],
    out_specs=[_ROWBLK, _HALLBLK],
    out_shape=[jax.ShapeDtypeStruct((N, H), jnp.float32), _HALL_SHAPE],
)

_tc_layer = pl.pallas_call(
    _layer_body,
    grid=(GRID,),
    in_specs=[_ROWBLK, _AGGBLK] + _MLP_SPECS + [_full((R, H, H))],
    out_specs=[_ROWBLK, _HALLBLK],
    out_shape=[jax.ShapeDtypeStruct((N, H), jnp.float32), _HALL_SHAPE],
)

_tc_final = pl.pallas_call(
    _final_body,
    grid=(GRID,),
    in_specs=[_ROWBLK, _AGGBLK] + _MLP_SPECS,
    out_specs=_ROWBLK,
    out_shape=jax.ShapeDtypeStruct((N, H), jnp.float32),
)


def kernel(x, edge_index, edges_type, Win, b_in, Wrel, Wself, b_rel,
           Wu1, b_u1, Wu2, b_u2):
    src = edge_index[0].astype(jnp.int32)
    dst = edge_index[1].astype(jnp.int32)
    et = edges_type.astype(jnp.int32)

    gidx = et * N + src
    pad = E_PAD - E
    gidx_p = jnp.concatenate([gidx, jnp.zeros((pad,), jnp.int32)])
    gidx_p = gidx_p.reshape(NS, NCHUNK, CHUNK)
    cores = []
    for c in range(NC):
        loc = dst - c * HALFN
        loc = jnp.where((loc >= 0) & (loc < HALFN), loc, HALFN)
        loc = jnp.concatenate([loc, jnp.full((pad,), HALFN, jnp.int32)])
        cores.append(jnp.stack([gidx_p, loc.reshape(NS, NCHUNK, CHUNK)],
                               axis=2))
    idx2 = jnp.stack(cores)

    h, hall = _tc_init(x, Win, b_in.reshape(1, H), Wrel[0])
    for l in range(L):
        agg = _sc_scatter()(hall.reshape(R * N, H), idx2)
        args = (h, agg, Wself[l], b_rel[l].reshape(1, H), Wu1[l],
                b_u1[l].reshape(1, 2 * H), Wu2[l], b_u2[l].reshape(1, H))
        if l < L - 1:
            h, hall = _tc_layer(*args, Wrel[l + 1])
        else:
            h = _tc_final(*args)
    return h


# speedup vs baseline: 1.0455x; 1.0455x over previous
import functools

import jax
import jax.numpy as jnp
from jax import lax
from jax.experimental import pallas as pl
from jax.experimental.pallas import tpu as pltpu
from jax.experimental.pallas import tpu_sc as plsc

N = 10000
E = 320000
H = 128
R = 16
L = 10

NC = 2
NS = 16
HALFN = N // NC
CHUNK = 128
CAPT = E + 2 * CHUNK
NCHUNKT = CAPT // CHUNK
NJMAX = -(-NCHUNKT // NS)
AGG_ROWS = 5120
ROWS_PER_SUB = AGG_ROWS // NS

BN = 1000
GRID = N // BN


def _sc_body(hall, idx2, scal, out, idxring, rows, svm, aggs,
             gsems, ssems, isems):
    cid = lax.axis_index("c")
    sid = lax.axis_index("s")

    pltpu.sync_copy(scal.at[cid], svm)
    v = svm[...]
    m_c = v[0]
    off_c = v[1]
    nj = lax.div(m_c - sid + NS - 1, jnp.int32(NS))

    def zero_row(r, _):
        for j in range(H // 16):
            rows[0, r, pl.ds(j * 16, 16)] = jnp.zeros((16,), jnp.float32)
        return 0

    lax.fori_loop(0, CHUNK, zero_row, 0)
    base = sid * ROWS_PER_SUB
    off = 0
    while off < ROWS_PER_SUB:
        step = min(CHUNK, ROWS_PER_SUB - off)
        pltpu.sync_copy(rows.at[0, pl.ds(0, step)],
                        aggs.at[pl.ds(base + off, step)])
        off += step

    plsc.subcore_barrier()

    def cg(j):
        return off_c + sid + NS * j

    for k in range(4):
        @pl.when(k < nj)
        def _pro_idx(k=k):
            pltpu.async_copy(idx2.at[cg(k)], idxring.at[k], isems.at[k])
    for k in range(2):
        @pl.when(k < nj)
        def _pro_gather(k=k):
            pltpu.make_async_copy(idx2.at[cg(k)], idxring.at[k],
                                  isems.at[k]).wait()
            pltpu.async_copy(hall.at[idxring.at[k, 0]], rows.at[k],
                             gsems.at[k])

    def chunk_step(c, _):
        @pl.when(c < nj)
        def _active():
            si = lax.rem(c, 8)
            sr = lax.rem(c, 4)
            sg = lax.rem(c, 2)
            pltpu.make_async_copy(hall.at[idxring.at[si, 0]], rows.at[sr],
                                  gsems.at[sg]).wait()

            @pl.when(c >= 2)
            def _drain_scatter():
                pltpu.make_async_copy(
                    rows.at[lax.rem(c + 2, 4)],
                    aggs.at[idxring.at[lax.rem(c + 6, 8), 1]],
                    ssems.at[sg]).wait()

            pltpu.async_copy(rows.at[sr], aggs.at[idxring.at[si, 1]],
                             ssems.at[sg], add=True)

            @pl.when(c + 2 < nj)
            def _next_gather():
                pltpu.make_async_copy(idx2.at[cg(c + 2)],
                                      idxring.at[lax.rem(c + 2, 8)],
                                      isems.at[lax.rem(c + 2, 4)]).wait()
                pltpu.async_copy(hall.at[idxring.at[lax.rem(c + 2, 8), 0]],
                                 rows.at[lax.rem(c + 2, 4)], gsems.at[sg])

            @pl.when(c + 4 < nj)
            def _next_idx():
                pltpu.async_copy(idx2.at[cg(c + 4)],
                                 idxring.at[lax.rem(c + 4, 8)],
                                 isems.at[lax.rem(c + 4, 4)])

        return 0

    lax.fori_loop(0, NJMAX, chunk_step, 0)

    for t in (2, 1):
        @pl.when(nj >= t)
        def _drain_tail(t=t):
            c = nj - t
            pltpu.make_async_copy(rows.at[lax.rem(c, 4)],
                                  aggs.at[idxring.at[lax.rem(c, 8), 1]],
                                  ssems.at[lax.rem(c, 2)]).wait()

    plsc.subcore_barrier()

    pltpu.sync_copy(aggs.at[pl.ds(base, ROWS_PER_SUB)],
                    out.at[cid, pl.ds(base, ROWS_PER_SUB)])


@functools.cache
def _sc_scatter():
    return functools.partial(
        pl.kernel,
        mesh=plsc.VectorSubcoreMesh(core_axis_name="c", subcore_axis_name="s"),
        out_type=jax.ShapeDtypeStruct((NC, AGG_ROWS, H), jnp.float32),
        scratch_types=[
            pltpu.VMEM((8, 2, CHUNK), jnp.int32),
            pltpu.VMEM((4, CHUNK, H), jnp.float32),
            pltpu.VMEM((16,), jnp.int32),
            pltpu.VMEM_SHARED((AGG_ROWS, H), jnp.float32),
            pltpu.SemaphoreType.DMA((2,)),
            pltpu.SemaphoreType.DMA((2,)),
            pltpu.SemaphoreType.DMA((4,)),
        ],
    )(_sc_body)


def _relu(v):
    return jnp.maximum(v, 0.0)


def _mm(a, b):
    return jnp.dot(a, b, preferred_element_type=jnp.float32)


def _init_body(x, win, b_in, wrel, h_ref, hall_ref):
    h = _relu(_mm(x[...], win[...]) + b_in[0][None, :])
    h_ref[...] = h
    for r in range(R):
        hall_ref[r] = _mm(h, wrel[r])


def _mlp(hb, agg, wself, b_rel, wu1, b_u1, wu2, b_u2):
    mid = agg[0] + _mm(hb, wself[...]) + b_rel[0][None, :]
    z = _relu(_mm(hb, wu1[0:H, :]) + _mm(mid, wu1[H:2 * H, :])
              + b_u1[0][None, :])
    return _relu(_mm(z, wu2[...]) + b_u2[0][None, :])


def _layer_body(h, agg, wself, b_rel, wu1, b_u1, wu2, b_u2, wrel,
                h_ref, hall_ref):
    out = _mlp(h[...], agg, wself, b_rel, wu1, b_u1, wu2, b_u2)
    h_ref[...] = out
    for r in range(R):
        hall_ref[r] = _mm(out, wrel[r])


def _final_body(h, agg, wself, b_rel, wu1, b_u1, wu2, b_u2, h_ref):
    h_ref[...] = _mlp(h[...], agg, wself, b_rel, wu1, b_u1, wu2, b_u2)


def _full(shape):
    return pl.BlockSpec(shape, lambda i: tuple(0 for _ in shape))


_ROWBLK = pl.BlockSpec((BN, H), lambda i: (i, 0))
_HALLBLK = pl.BlockSpec((R, BN, H), lambda i: (0, i, 0))
_AGGBLK = pl.BlockSpec((1, BN, H), lambda i: (i // 5, i % 5, 0))
_HALL_SHAPE = jax.ShapeDtypeStruct((R, N, H), jnp.float32)

_MLP_SPECS = [_full((H, H)), _full((1, H)), _full((2 * H, 2 * H)),
              _full((1, 2 * H)), _full((2 * H, H)), _full((1, H))]

_tc_init = pl.pallas_call(
    _init_body,
    grid=(GRID,),
    in_specs=[_ROWBLK, _full((H, H)), _full((1, H)), _full((R, H, H))],
    out_specs=[_ROWBLK, _HALLBLK],
    out_shape=[jax.ShapeDtypeStruct((N, H), jnp.float32), _HALL_SHAPE],
)

_tc_layer = pl.pallas_call(
    _layer_body,
    grid=(GRID,),
    in_specs=[_ROWBLK, _AGGBLK] + _MLP_SPECS + [_full((R, H, H))],
    out_specs=[_ROWBLK, _HALLBLK],
    out_shape=[jax.ShapeDtypeStruct((N, H), jnp.float32), _HALL_SHAPE],
)

_tc_final = pl.pallas_call(
    _final_body,
    grid=(GRID,),
    in_specs=[_ROWBLK, _AGGBLK] + _MLP_SPECS,
    out_specs=_ROWBLK,
    out_shape=jax.ShapeDtypeStruct((N, H), jnp.float32),
)


def kernel(x, edge_index, edges_type, Win, b_in, Wrel, Wself, b_rel,
           Wu1, b_u1, Wu2, b_u2):
    src = edge_index[0].astype(jnp.int32)
    dst = edge_index[1].astype(jnp.int32)
    et = edges_type.astype(jnp.int32)

    gidx = et * N + src

    b = (dst >= HALFN).astype(jnp.int32)
    dstl = dst - b * HALFN
    c1 = jnp.cumsum(b)
    n1 = c1[-1]
    n0 = E - n1
    m0 = (n0 + CHUNK - 1) // CHUNK
    m1 = (n1 + CHUNK - 1) // CHUNK
    o1 = m0 * CHUNK
    i = jnp.arange(E, dtype=jnp.int32)
    flat = jnp.where(b == 1, o1 + c1 - 1, i - c1)
    g_arr = jnp.zeros((CAPT,), jnp.int32).at[flat].set(
        gidx, unique_indices=True)
    d_arr = jnp.full((CAPT,), HALFN, jnp.int32).at[flat].set(
        dstl, unique_indices=True)
    idx2 = jnp.stack([g_arr.reshape(NCHUNKT, CHUNK),
                      d_arr.reshape(NCHUNKT, CHUNK)], axis=1)
    zero = jnp.zeros((), jnp.int32)
    scal = jnp.stack([jnp.stack([m0, zero] + [zero] * 14),
                      jnp.stack([m1, m0] + [zero] * 14)])

    h, hall = _tc_init(x, Win, b_in.reshape(1, H), Wrel[0])
    for l in range(L):
        agg = _sc_scatter()(hall.reshape(R * N, H), idx2, scal)
        args = (h, agg, Wself[l], b_rel[l].reshape(1, H), Wu1[l],
                b_u1[l].reshape(1, 2 * H), Wu2[l], b_u2[l].reshape(1, H))
        if l < L - 1:
            h, hall = _tc_layer(*args, Wrel[l + 1])
        else:
            h = _tc_final(*args)
    return h


# speedup vs baseline: 1.5487x; 1.4814x over previous
import functools

import jax
import jax.numpy as jnp
from jax import lax
from jax.experimental import pallas as pl
from jax.experimental.pallas import tpu as pltpu
from jax.experimental.pallas import tpu_sc as plsc

N = 10000
E = 320000
H = 128
R = 16
L = 10

NC = 2
NS = 16
HALFN = N // NC
CHUNK = 128
CAPT = E + 2 * CHUNK
NCHUNKT = CAPT // CHUNK
NJMAX = -(-NCHUNKT // NS)
AGG_ROWS = 5120
ROWS_PER_SUB = AGG_ROWS // NS

BN = 1000
GRID = N // BN

EP = E + 2 * CHUNK
NCHP = EP // CHUNK
NJP = -(-NCHP // (NC * NS))
OBASE = NCHUNKT * 2 * CHUNK
OSIZE = OBASE + 4 * CHUNK


def _sc_body(hall, idx2, scal, out, idxring, rows, svm, aggs,
             gsems, ssems, isems):
    cid = lax.axis_index("c")
    sid = lax.axis_index("s")

    pltpu.sync_copy(scal.at[cid], svm)
    v = svm[...]
    m_c = v[0]
    off_c = v[1]
    nj = lax.div(m_c - sid + NS - 1, jnp.int32(NS))

    def zero_row(r, _):
        for j in range(H // 16):
            rows[0, r, pl.ds(j * 16, 16)] = jnp.zeros((16,), jnp.float32)
        return 0

    lax.fori_loop(0, CHUNK, zero_row, 0)
    base = sid * ROWS_PER_SUB
    off = 0
    while off < ROWS_PER_SUB:
        step = min(CHUNK, ROWS_PER_SUB - off)
        pltpu.sync_copy(rows.at[0, pl.ds(0, step)],
                        aggs.at[pl.ds(base + off, step)])
        off += step

    plsc.subcore_barrier()

    def cg(j):
        return off_c + sid + NS * j

    for k in range(4):
        @pl.when(k < nj)
        def _pro_idx(k=k):
            pltpu.async_copy(idx2.at[cg(k)], idxring.at[k], isems.at[k])
    for k in range(2):
        @pl.when(k < nj)
        def _pro_gather(k=k):
            pltpu.make_async_copy(idx2.at[cg(k)], idxring.at[k],
                                  isems.at[k]).wait()
            pltpu.async_copy(hall.at[idxring.at[k, 0]], rows.at[k],
                             gsems.at[k])

    def chunk_step(c, _):
        @pl.when(c < nj)
        def _active():
            si = lax.rem(c, 8)
            sr = lax.rem(c, 4)
            sg = lax.rem(c, 2)
            pltpu.make_async_copy(hall.at[idxring.at[si, 0]], rows.at[sr],
                                  gsems.at[sg]).wait()

            @pl.when(c >= 2)
            def _drain_scatter():
                pltpu.make_async_copy(
                    rows.at[lax.rem(c + 2, 4)],
                    aggs.at[idxring.at[lax.rem(c + 6, 8), 1]],
                    ssems.at[sg]).wait()

            pltpu.async_copy(rows.at[sr], aggs.at[idxring.at[si, 1]],
                             ssems.at[sg], add=True)

            @pl.when(c + 2 < nj)
            def _next_gather():
                pltpu.make_async_copy(idx2.at[cg(c + 2)],
                                      idxring.at[lax.rem(c + 2, 8)],
                                      isems.at[lax.rem(c + 2, 4)]).wait()
                pltpu.async_copy(hall.at[idxring.at[lax.rem(c + 2, 8), 0]],
                                 rows.at[lax.rem(c + 2, 4)], gsems.at[sg])

            @pl.when(c + 4 < nj)
            def _next_idx():
                pltpu.async_copy(idx2.at[cg(c + 4)],
                                 idxring.at[lax.rem(c + 4, 8)],
                                 isems.at[lax.rem(c + 4, 4)])

        return 0

    lax.fori_loop(0, NJMAX, chunk_step, 0)

    for t in (2, 1):
        @pl.when(nj >= t)
        def _drain_tail(t=t):
            c = nj - t
            pltpu.make_async_copy(rows.at[lax.rem(c, 4)],
                                  aggs.at[idxring.at[lax.rem(c, 8), 1]],
                                  ssems.at[lax.rem(c, 2)]).wait()

    plsc.subcore_barrier()

    pltpu.sync_copy(aggs.at[pl.ds(base, ROWS_PER_SUB)],
                    out.at[cid, pl.ds(base, ROWS_PER_SUB)])


def _sc_part_body(vals, oidx, bufs, lsems, g1sems, g2sems):
    cid = lax.axis_index("c")
    sid = lax.axis_index("s")
    w = sid * NC + cid

    def q(j):
        return w + (NC * NS) * j

    njw = lax.div(jnp.int32(NCHP) - w + NC * NS - 1, jnp.int32(NC * NS))

    for k in range(2):
        @pl.when(k < njw)
        def _pro(k=k):
            pltpu.async_copy(vals.at[q(k)], bufs.at[k], lsems.at[k])

    def step(j, _):
        @pl.when(j < njw)
        def _active():
            b = lax.rem(j, 4)
            s = lax.rem(j, 2)
            pltpu.make_async_copy(vals.at[q(j)], bufs.at[b],
                                  lsems.at[s]).wait()

            @pl.when(j >= 2)
            def _drain():
                b2 = lax.rem(j + 2, 4)
                pltpu.make_async_copy(bufs.at[b2, 0],
                                      oidx.at[bufs.at[b2, 2]],
                                      g1sems.at[s]).wait()
                pltpu.make_async_copy(bufs.at[b2, 1],
                                      oidx.at[bufs.at[b2, 3]],
                                      g2sems.at[s]).wait()

            pltpu.async_copy(bufs.at[b, 0], oidx.at[bufs.at[b, 2]],
                             g1sems.at[s])
            pltpu.async_copy(bufs.at[b, 1], oidx.at[bufs.at[b, 3]],
                             g2sems.at[s])

            @pl.when(j + 2 < njw)
            def _next():
                pltpu.async_copy(vals.at[q(j + 2)],
                                 bufs.at[lax.rem(j + 2, 4)], lsems.at[s])

        return 0

    lax.fori_loop(0, NJP, step, 0)

    for t in (2, 1):
        @pl.when(njw >= t)
        def _tail(t=t):
            j = njw - t
            b = lax.rem(j, 4)
            s = lax.rem(j, 2)
            pltpu.make_async_copy(bufs.at[b, 0], oidx.at[bufs.at[b, 2]],
                                  g1sems.at[s]).wait()
            pltpu.make_async_copy(bufs.at[b, 1], oidx.at[bufs.at[b, 3]],
                                  g2sems.at[s]).wait()


@functools.cache
def _sc_partition():
    return functools.partial(
        pl.kernel,
        mesh=plsc.VectorSubcoreMesh(core_axis_name="c", subcore_axis_name="s"),
        out_type=jax.ShapeDtypeStruct((OSIZE,), jnp.int32),
        scratch_types=[
            pltpu.VMEM((4, 4, CHUNK), jnp.int32),
            pltpu.SemaphoreType.DMA((2,)),
            pltpu.SemaphoreType.DMA((2,)),
            pltpu.SemaphoreType.DMA((2,)),
        ],
    )(_sc_part_body)


@functools.cache
def _sc_scatter():
    return functools.partial(
        pl.kernel,
        mesh=plsc.VectorSubcoreMesh(core_axis_name="c", subcore_axis_name="s"),
        out_type=jax.ShapeDtypeStruct((NC, AGG_ROWS, H), jnp.float32),
        scratch_types=[
            pltpu.VMEM((8, 2, CHUNK), jnp.int32),
            pltpu.VMEM((4, CHUNK, H), jnp.float32),
            pltpu.VMEM((16,), jnp.int32),
            pltpu.VMEM_SHARED((AGG_ROWS, H), jnp.float32),
            pltpu.SemaphoreType.DMA((2,)),
            pltpu.SemaphoreType.DMA((2,)),
            pltpu.SemaphoreType.DMA((4,)),
        ],
    )(_sc_body)


def _relu(v):
    return jnp.maximum(v, 0.0)


def _mm(a, b):
    return jnp.dot(a, b, preferred_element_type=jnp.float32)


def _init_body(x, win, b_in, wrel, h_ref, hall_ref):
    h = _relu(_mm(x[...], win[...]) + b_in[0][None, :])
    h_ref[...] = h
    for r in range(R):
        hall_ref[r] = _mm(h, wrel[r])


def _mlp(hb, agg, wself, b_rel, wu1, b_u1, wu2, b_u2):
    mid = agg[0] + _mm(hb, wself[...]) + b_rel[0][None, :]
    z = _relu(_mm(hb, wu1[0:H, :]) + _mm(mid, wu1[H:2 * H, :])
              + b_u1[0][None, :])
    return _relu(_mm(z, wu2[...]) + b_u2[0][None, :])


def _layer_body(h, agg, wself, b_rel, wu1, b_u1, wu2, b_u2, wrel,
                h_ref, hall_ref):
    out = _mlp(h[...], agg, wself, b_rel, wu1, b_u1, wu2, b_u2)
    h_ref[...] = out
    for r in range(R):
        hall_ref[r] = _mm(out, wrel[r])


def _final_body(h, agg, wself, b_rel, wu1, b_u1, wu2, b_u2, h_ref):
    h_ref[...] = _mlp(h[...], agg, wself, b_rel, wu1, b_u1, wu2, b_u2)


def _full(shape):
    return pl.BlockSpec(shape, lambda i: tuple(0 for _ in shape))


_ROWBLK = pl.BlockSpec((BN, H), lambda i: (i, 0))
_HALLBLK = pl.BlockSpec((R, BN, H), lambda i: (0, i, 0))
_AGGBLK = pl.BlockSpec((1, BN, H), lambda i: (i // 5, i % 5, 0))
_HALL_SHAPE = jax.ShapeDtypeStruct((R, N, H), jnp.float32)

_MLP_SPECS = [_full((H, H)), _full((1, H)), _full((2 * H, 2 * H)),
              _full((1, 2 * H)), _full((2 * H, H)), _full((1, H))]

_tc_init = pl.pallas_call(
    _init_body,
    grid=(GRID,),
    in_specs=[_ROWBLK, _full((H, H)), _full((1, H)), _full((R, H, H))],
    out_specs=[_ROWBLK, _HALLBLK],
    out_shape=[jax.ShapeDtypeStruct((N, H), jnp.float32), _HALL_SHAPE],
)

_tc_layer = pl.pallas_call(
    _layer_body,
    grid=(GRID,),
    in_specs=[_ROWBLK, _AGGBLK] + _MLP_SPECS + [_full((R, H, H))],
    out_specs=[_ROWBLK, _HALLBLK],
    out_shape=[jax.ShapeDtypeStruct((N, H), jnp.float32), _HALL_SHAPE],
)

_tc_final = pl.pallas_call(
    _final_body,
    grid=(GRID,),
    in_specs=[_ROWBLK, _AGGBLK] + _MLP_SPECS,
    out_specs=_ROWBLK,
    out_shape=jax.ShapeDtypeStruct((N, H), jnp.float32),
)


def kernel(x, edge_index, edges_type, Win, b_in, Wrel, Wself, b_rel,
           Wu1, b_u1, Wu2, b_u2):
    src = edge_index[0].astype(jnp.int32)
    dst = edge_index[1].astype(jnp.int32)
    et = edges_type.astype(jnp.int32)

    gidx = et * N + src

    b = (dst >= HALFN).astype(jnp.int32)
    dstl = dst - b * HALFN
    c1 = jnp.cumsum(b)
    n1 = c1[-1]
    n0 = E - n1
    m0 = (n0 + CHUNK - 1) // CHUNK
    m1 = (n1 + CHUNK - 1) // CHUNK
    o1 = m0 * CHUNK
    i = jnp.arange(E, dtype=jnp.int32)
    flat = jnp.where(b == 1, o1 + c1 - 1, i - c1)
    gpos_e = (flat // CHUNK) * (2 * CHUNK) + flat % CHUNK
    k = jnp.arange(CHUNK, dtype=jnp.int32)
    sA = n0 + k
    tA = (sA // CHUNK) * (2 * CHUNK) + sA % CHUNK
    inA = sA < o1
    gposA = jnp.where(inA, tA, OBASE + k)
    dposA = jnp.where(inA, tA + CHUNK, OBASE + CHUNK + k)
    sB = o1 + n1 + k
    tB = (sB // CHUNK) * (2 * CHUNK) + sB % CHUNK
    inB = sB < o1 + m1 * CHUNK
    gposB = jnp.where(inB, tB, OBASE + 2 * CHUNK + k)
    dposB = jnp.where(inB, tB + CHUNK, OBASE + 3 * CHUNK + k)
    padv = jnp.zeros((2 * CHUNK,), jnp.int32)
    gv = jnp.concatenate([gidx, padv])
    dv = jnp.concatenate([dstl, padv + HALFN])
    gp = jnp.concatenate([gpos_e, gposA, gposB])
    dp = jnp.concatenate([gpos_e + CHUNK, dposA, dposB])
    vals = jnp.stack([gv.reshape(NCHP, CHUNK), dv.reshape(NCHP, CHUNK),
                      gp.reshape(NCHP, CHUNK), dp.reshape(NCHP, CHUNK)],
                     axis=1)
    oidx = _sc_partition()(vals)
    idx2 = oidx.reshape(NCHUNKT + 2, 2, CHUNK)
    zero = jnp.zeros((), jnp.int32)
    scal = jnp.stack([jnp.stack([m0, zero] + [zero] * 14),
                      jnp.stack([m1, m0] + [zero] * 14)])

    h, hall = _tc_init(x, Win, b_in.reshape(1, H), Wrel[0])
    for l in range(L):
        agg = _sc_scatter()(hall.reshape(R * N, H), idx2, scal)
        args = (h, agg, Wself[l], b_rel[l].reshape(1, H), Wu1[l],
                b_u1[l].reshape(1, 2 * H), Wu2[l], b_u2[l].reshape(1, H))
        if l < L - 1:
            h, hall = _tc_layer(*args, Wrel[l + 1])
        else:
            h = _tc_final(*args)
    return h


# speedup vs baseline: 1.5506x; 1.0012x over previous
import functools

import jax
import jax.numpy as jnp
from jax import lax
from jax.experimental import pallas as pl
from jax.experimental.pallas import tpu as pltpu
from jax.experimental.pallas import tpu_sc as plsc

N = 10000
E = 320000
H = 128
R = 16
L = 10

NC = 2
NS = 16
HALFN = N // NC
CHUNK = 128
CAPT = E + 2 * CHUNK
NCHUNKT = CAPT // CHUNK
NJMAX = -(-NCHUNKT // NS)
AGG_ROWS = 5120
ROWS_PER_SUB = AGG_ROWS // NS

BN = 1000
GRID = N // BN

EP = E + 2 * CHUNK
NCHP = EP // CHUNK
NJP = -(-NCHP // (NC * NS))
OBASE = NCHUNKT * 2 * CHUNK
OSIZE = OBASE + 4 * CHUNK


def _sc_body(hall, idx2, scal, out, idxring, rows, svm, aggs,
             gsems, ssems, isems):
    cid = lax.axis_index("c")
    sid = lax.axis_index("s")

    pltpu.sync_copy(scal.at[cid], svm)
    v = svm[...]
    m_c = v[0]
    off_c = v[1]
    nj = lax.div(m_c - sid + NS - 1, jnp.int32(NS))

    def zero_row(r, _):
        for j in range(H // 16):
            rows[0, r, pl.ds(j * 16, 16)] = jnp.zeros((16,), jnp.float32)
        return 0

    lax.fori_loop(0, CHUNK, zero_row, 0)
    base = sid * ROWS_PER_SUB
    off = 0
    while off < ROWS_PER_SUB:
        step = min(CHUNK, ROWS_PER_SUB - off)
        pltpu.sync_copy(rows.at[0, pl.ds(0, step)],
                        aggs.at[pl.ds(base + off, step)])
        off += step

    plsc.subcore_barrier()

    def cg(j):
        return off_c + sid + NS * j

    for k in range(4):
        @pl.when(k < nj)
        def _pro_idx(k=k):
            pltpu.async_copy(idx2.at[cg(k)], idxring.at[k], isems.at[k])
    for k in range(2):
        @pl.when(k < nj)
        def _pro_gather(k=k):
            pltpu.make_async_copy(idx2.at[cg(k)], idxring.at[k],
                                  isems.at[k]).wait()
            pltpu.async_copy(hall.at[idxring.at[k, 0]], rows.at[k],
                             gsems.at[k])

    def chunk_step(c, _):
        @pl.when(c < nj)
        def _active():
            si = lax.rem(c, 8)
            sr = lax.rem(c, 4)
            sg = lax.rem(c, 2)
            pltpu.make_async_copy(hall.at[idxring.at[si, 0]], rows.at[sr],
                                  gsems.at[sg]).wait()

            @pl.when(c >= 2)
            def _drain_scatter():
                pltpu.make_async_copy(
                    rows.at[lax.rem(c + 2, 4)],
                    aggs.at[idxring.at[lax.rem(c + 6, 8), 1]],
                    ssems.at[sg]).wait()

            pltpu.async_copy(rows.at[sr], aggs.at[idxring.at[si, 1]],
                             ssems.at[sg], add=True)

            @pl.when(c + 2 < nj)
            def _next_gather():
                pltpu.make_async_copy(idx2.at[cg(c + 2)],
                                      idxring.at[lax.rem(c + 2, 8)],
                                      isems.at[lax.rem(c + 2, 4)]).wait()
                pltpu.async_copy(hall.at[idxring.at[lax.rem(c + 2, 8), 0]],
                                 rows.at[lax.rem(c + 2, 4)], gsems.at[sg])

            @pl.when(c + 4 < nj)
            def _next_idx():
                pltpu.async_copy(idx2.at[cg(c + 4)],
                                 idxring.at[lax.rem(c + 4, 8)],
                                 isems.at[lax.rem(c + 4, 4)])

        return 0

    lax.fori_loop(0, NJMAX, chunk_step, 0)

    for t in (2, 1):
        @pl.when(nj >= t)
        def _drain_tail(t=t):
            c = nj - t
            pltpu.make_async_copy(rows.at[lax.rem(c, 4)],
                                  aggs.at[idxring.at[lax.rem(c, 8), 1]],
                                  ssems.at[lax.rem(c, 2)]).wait()

    plsc.subcore_barrier()

    pltpu.sync_copy(aggs.at[pl.ds(base, ROWS_PER_SUB)],
                    out.at[cid, pl.ds(base, ROWS_PER_SUB)])


def _sc_part_body(vals, oidx, bufs, lsems, g1sems, g2sems):
    cid = lax.axis_index("c")
    sid = lax.axis_index("s")
    w = sid * NC + cid

    def q(j):
        return w + (NC * NS) * j

    njw = lax.div(jnp.int32(NCHP) - w + NC * NS - 1, jnp.int32(NC * NS))

    for k in range(4):
        @pl.when(k < njw)
        def _pro(k=k):
            pltpu.async_copy(vals.at[q(k)], bufs.at[k], lsems.at[k])

    def step(j, _):
        @pl.when(j < njw)
        def _active():
            b = lax.rem(j, 8)
            s = lax.rem(j, 4)
            pltpu.make_async_copy(vals.at[q(j)], bufs.at[b],
                                  lsems.at[s]).wait()

            @pl.when(j >= 4)
            def _drain():
                b2 = lax.rem(j + 4, 8)
                pltpu.make_async_copy(bufs.at[b2, 0],
                                      oidx.at[bufs.at[b2, 2]],
                                      g1sems.at[s]).wait()
                pltpu.make_async_copy(bufs.at[b2, 1],
                                      oidx.at[bufs.at[b2, 3]],
                                      g2sems.at[s]).wait()

            pltpu.async_copy(bufs.at[b, 0], oidx.at[bufs.at[b, 2]],
                             g1sems.at[s])
            pltpu.async_copy(bufs.at[b, 1], oidx.at[bufs.at[b, 3]],
                             g2sems.at[s])

            @pl.when(j + 4 < njw)
            def _next():
                pltpu.async_copy(vals.at[q(j + 4)],
                                 bufs.at[lax.rem(j + 4, 8)], lsems.at[s])

        return 0

    lax.fori_loop(0, NJP, step, 0)

    for t in (4, 3, 2, 1):
        @pl.when(njw >= t)
        def _tail(t=t):
            j = njw - t
            b = lax.rem(j, 8)
            s = lax.rem(j, 4)
            pltpu.make_async_copy(bufs.at[b, 0], oidx.at[bufs.at[b, 2]],
                                  g1sems.at[s]).wait()
            pltpu.make_async_copy(bufs.at[b, 1], oidx.at[bufs.at[b, 3]],
                                  g2sems.at[s]).wait()


@functools.cache
def _sc_partition():
    return functools.partial(
        pl.kernel,
        mesh=plsc.VectorSubcoreMesh(core_axis_name="c", subcore_axis_name="s"),
        out_type=jax.ShapeDtypeStruct((OSIZE,), jnp.int32),
        scratch_types=[
            pltpu.VMEM((8, 4, CHUNK), jnp.int32),
            pltpu.SemaphoreType.DMA((4,)),
            pltpu.SemaphoreType.DMA((4,)),
            pltpu.SemaphoreType.DMA((4,)),
        ],
    )(_sc_part_body)


@functools.cache
def _sc_scatter():
    return functools.partial(
        pl.kernel,
        mesh=plsc.VectorSubcoreMesh(core_axis_name="c", subcore_axis_name="s"),
        out_type=jax.ShapeDtypeStruct((NC, AGG_ROWS, H), jnp.float32),
        scratch_types=[
            pltpu.VMEM((8, 2, CHUNK), jnp.int32),
            pltpu.VMEM((4, CHUNK, H), jnp.float32),
            pltpu.VMEM((16,), jnp.int32),
            pltpu.VMEM_SHARED((AGG_ROWS, H), jnp.float32),
            pltpu.SemaphoreType.DMA((2,)),
            pltpu.SemaphoreType.DMA((2,)),
            pltpu.SemaphoreType.DMA((4,)),
        ],
    )(_sc_body)


def _relu(v):
    return jnp.maximum(v, 0.0)


def _mm(a, b):
    return jnp.dot(a, b, preferred_element_type=jnp.float32)


def _init_body(x, win, b_in, wrel, h_ref, hall_ref):
    h = _relu(_mm(x[...], win[...]) + b_in[0][None, :])
    h_ref[...] = h
    for r in range(R):
        hall_ref[r] = _mm(h, wrel[r])


def _mlp(hb, agg, wself, b_rel, wu1, b_u1, wu2, b_u2):
    mid = agg[0] + _mm(hb, wself[...]) + b_rel[0][None, :]
    z = _relu(_mm(hb, wu1[0:H, :]) + _mm(mid, wu1[H:2 * H, :])
              + b_u1[0][None, :])
    return _relu(_mm(z, wu2[...]) + b_u2[0][None, :])


def _layer_body(h, agg, wself, b_rel, wu1, b_u1, wu2, b_u2, wrel,
                h_ref, hall_ref):
    out = _mlp(h[...], agg, wself, b_rel, wu1, b_u1, wu2, b_u2)
    h_ref[...] = out
    for r in range(R):
        hall_ref[r] = _mm(out, wrel[r])


def _final_body(h, agg, wself, b_rel, wu1, b_u1, wu2, b_u2, h_ref):
    h_ref[...] = _mlp(h[...], agg, wself, b_rel, wu1, b_u1, wu2, b_u2)


def _full(shape):
    return pl.BlockSpec(shape, lambda i: tuple(0 for _ in shape))


_ROWBLK = pl.BlockSpec((BN, H), lambda i: (i, 0))
_HALLBLK = pl.BlockSpec((R, BN, H), lambda i: (0, i, 0))
_AGGBLK = pl.BlockSpec((1, BN, H), lambda i: (i // 5, i % 5, 0))
_HALL_SHAPE = jax.ShapeDtypeStruct((R, N, H), jnp.float32)

_MLP_SPECS = [_full((H, H)), _full((1, H)), _full((2 * H, 2 * H)),
              _full((1, 2 * H)), _full((2 * H, H)), _full((1, H))]

_tc_init = pl.pallas_call(
    _init_body,
    grid=(GRID,),
    in_specs=[_ROWBLK, _full((H, H)), _full((1, H)), _full((R, H, H))],
    out_specs=[_ROWBLK, _HALLBLK],
    out_shape=[jax.ShapeDtypeStruct((N, H), jnp.float32), _HALL_SHAPE],
)

_tc_layer = pl.pallas_call(
    _layer_body,
    grid=(GRID,),
    in_specs=[_ROWBLK, _AGGBLK] + _MLP_SPECS + [_full((R, H, H))],
    out_specs=[_ROWBLK, _HALLBLK],
    out_shape=[jax.ShapeDtypeStruct((N, H), jnp.float32), _HALL_SHAPE],
)

_tc_final = pl.pallas_call(
    _final_body,
    grid=(GRID,),
    in_specs=[_ROWBLK, _AGGBLK] + _MLP_SPECS,
    out_specs=_ROWBLK,
    out_shape=jax.ShapeDtypeStruct((N, H), jnp.float32),
)


def kernel(x, edge_index, edges_type, Win, b_in, Wrel, Wself, b_rel,
           Wu1, b_u1, Wu2, b_u2):
    src = edge_index[0].astype(jnp.int32)
    dst = edge_index[1].astype(jnp.int32)
    et = edges_type.astype(jnp.int32)

    gidx = et * N + src

    b = (dst >= HALFN).astype(jnp.int32)
    dstl = dst - b * HALFN
    c1 = jnp.cumsum(b)
    n1 = c1[-1]
    n0 = E - n1
    m0 = (n0 + CHUNK - 1) // CHUNK
    m1 = (n1 + CHUNK - 1) // CHUNK
    o1 = m0 * CHUNK
    i = jnp.arange(E, dtype=jnp.int32)
    flat = jnp.where(b == 1, o1 + c1 - 1, i - c1)
    gpos_e = (flat // CHUNK) * (2 * CHUNK) + flat % CHUNK
    k = jnp.arange(CHUNK, dtype=jnp.int32)
    sA = n0 + k
    tA = (sA // CHUNK) * (2 * CHUNK) + sA % CHUNK
    inA = sA < o1
    gposA = jnp.where(inA, tA, OBASE + k)
    dposA = jnp.where(inA, tA + CHUNK, OBASE + CHUNK + k)
    sB = o1 + n1 + k
    tB = (sB // CHUNK) * (2 * CHUNK) + sB % CHUNK
    inB = sB < o1 + m1 * CHUNK
    gposB = jnp.where(inB, tB, OBASE + 2 * CHUNK + k)
    dposB = jnp.where(inB, tB + CHUNK, OBASE + 3 * CHUNK + k)
    padv = jnp.zeros((2 * CHUNK,), jnp.int32)
    gv = jnp.concatenate([gidx, padv])
    dv = jnp.concatenate([dstl, padv + HALFN])
    gp = jnp.concatenate([gpos_e, gposA, gposB])
    dp = jnp.concatenate([gpos_e + CHUNK, dposA, dposB])
    vals = jnp.stack([gv.reshape(NCHP, CHUNK), dv.reshape(NCHP, CHUNK),
                      gp.reshape(NCHP, CHUNK), dp.reshape(NCHP, CHUNK)],
                     axis=1)
    oidx = _sc_partition()(vals)
    idx2 = oidx.reshape(NCHUNKT + 2, 2, CHUNK)
    zero = jnp.zeros((), jnp.int32)
    scal = jnp.stack([jnp.stack([m0, zero] + [zero] * 14),
                      jnp.stack([m1, m0] + [zero] * 14)])

    h, hall = _tc_init(x, Win, b_in.reshape(1, H), Wrel[0])
    for l in range(L):
        agg = _sc_scatter()(hall.reshape(R * N, H), idx2, scal)
        args = (h, agg, Wself[l], b_rel[l].reshape(1, H), Wu1[l],
                b_u1[l].reshape(1, 2 * H), Wu2[l], b_u2[l].reshape(1, H))
        if l < L - 1:
            h, hall = _tc_layer(*args, Wrel[l + 1])
        else:
            h = _tc_final(*args)
    return h
